# baseline (speedup 1.0000x reference)
import functools

import jax
import jax.numpy as jnp
from jax import lax
from jax.experimental import pallas as pl
from jax.experimental.pallas import tpu as pltpu
from jax.experimental.pallas import tpu_sc as plsc

RES = 256
N = 1048576
NW = 32
P = N // NW
C = 256
G = C // 16
NCHUNK = P // C
CAP = 64
NROW8 = RES * RES * RES * 3 // 8

_mesh = plsc.VectorSubcoreMesh(core_axis_name="c", subcore_axis_name="s")


@functools.partial(
    pl.kernel,
    mesh=_mesh,
    out_type=tuple(jax.ShapeDtypeStruct((N,), jnp.float32) for _ in range(3)),
    compiler_params=pltpu.CompilerParams(
        use_tc_tiling_on_sc=False, needs_layout_passes=False),
    scratch_types=[
        tuple(tuple(pltpu.VMEM((C,), jnp.float32) for _ in range(3))
              for _ in range(2)),
        tuple(pltpu.VMEM((12 * C,), jnp.int32) for _ in range(2)),
        tuple(pltpu.VMEM((12 * C,), jnp.int32) for _ in range(2)),
        tuple(pltpu.VMEM((C,), jnp.int32) for _ in range(2)),
        pltpu.VMEM((2, C), jnp.int32),
        tuple(pltpu.VMEM((12 * C, 8), jnp.float32) for _ in range(2)),
        tuple(pltpu.VMEM((12 * C, 8), jnp.float32) for _ in range(2)),
        pltpu.VMEM((6, C), jnp.float32),
        tuple(pltpu.VMEM((C,), jnp.float32) for _ in range(3)),
        tuple(pltpu.SemaphoreType.DMA for _ in range(2)),
        tuple(pltpu.SemaphoreType.DMA for _ in range(2)),
    ],
)
def _trilerp(xs_hbm, ys_hbm, zs_hbm, tab8_hbm, ox_hbm, oy_hbm, oz_hbm,
             pts_v, idx_v, idx2_v, xid_v, cl_v, rows_v, rows2_v, t_v, out_v,
             sem, sem2):
    wid = lax.axis_index("s") * 2 + lax.axis_index("c")
    lanes = lax.iota(jnp.int32, 16)
    coord_hbm = (xs_hbm, ys_hbm, zs_hbm)
    o_hbm = (ox_hbm, oy_hbm, oz_hbm)

    def init_body(g, _):
        zero = jnp.zeros((16,), jnp.int32)
        for slot in range(2):
            idx2_v[slot][pl.ds(g * 16, 16)] = zero
        return 0

    lax.fori_loop(0, 12 * G, init_body, 0)

    def initx_body(g, _):
        zero = jnp.zeros((16,), jnp.int32)
        for slot in range(2):
            xid_v[slot][pl.ds(g * 16, 16)] = zero
        return 0

    lax.fori_loop(0, G, initx_body, 0)

    def load_pts(i, slot):
        base = wid * P + i * C
        for ch in range(3):
            pltpu.sync_copy(coord_hbm[ch].at[pl.ds(base, C)],
                            pts_v[slot][ch])

    def do_idx(i, slot):
        def idx_body(g, off):
            g16 = g * 16
            p = g16 + lanes
            lo = []
            for ch in range(3):
                coord = pts_v[slot][ch][pl.ds(g16, 16)]
                s = coord * jnp.float32(RES - 1)
                li = jnp.minimum(s.astype(jnp.int32), RES - 2)
                t_v[slot * 3 + ch, pl.ds(g16, 16)] = (
                    s - li.astype(jnp.float32))
                lo.append(li)
            ix, iy, iz = lo
            izc = iz * 3
            iy1 = iy + 1
            ix1 = ix + 1
            yt = (((iy >> 3) << 8) + ((iy & 7) << 4),
                  ((iy1 >> 3) << 8) + ((iy1 & 7) << 4))
            xt_lo = ((ix >> 7) << 7) + ((ix >> 3) & 15)
            xt_hi = ((ix1 >> 7) << 7) + ((ix1 >> 3) & 15)
            cl = ix & 7
            cl_v[slot, pl.ds(g16, 16)] = cl
            msk = cl == 7
            mi = msk.astype(jnp.int32)
            rank = off + plsc.cumsum(mi) - 1
            ncross = jnp.sum(mi)
            plsc.store_scatter(xid_v[slot], [rank], p, mask=msk)
            yx = ((yt[0] + xt_lo, yt[0] + xt_hi),
                  (yt[1] + xt_lo, yt[1] + xt_hi))
            r12 = rank * 12
            for dz in range(2):
                for ch in range(3):
                    zterm = (izc + (dz * 3 + ch)) << 13
                    for dy in range(2):
                        m12 = (dz * 3 + ch) * 2 + dy
                        idx_v[slot][pl.ds(m12 * C + g16, 16)] = (
                            zterm + yx[dy][0])
                        plsc.store_scatter(idx2_v[slot], [r12 + m12],
                                           zterm + yx[dy][1], mask=msk)
            return off + ncross

        return lax.fori_loop(0, G, idx_body, jnp.int32(0))

    def fire(slot, nx):
        pltpu.async_copy(tab8_hbm.at[idx_v[slot]], rows_v[slot], sem[slot])

        @pl.when(jnp.logical_and(nx > 0, nx <= CAP))
        def _():
            pltpu.async_copy(
                tab8_hbm.at[idx2_v[slot].at[pl.ds(0, 12 * CAP)]],
                rows2_v[slot].at[pl.ds(0, 12 * CAP)], sem2[slot])

        @pl.when(nx > CAP)
        def _():
            pltpu.async_copy(
                tab8_hbm.at[idx2_v[slot]], rows2_v[slot], sem2[slot])

    def wait_gather(slot, nx):
        pltpu.make_async_copy(
            tab8_hbm.at[idx_v[slot]], rows_v[slot], sem[slot]).wait()

        @pl.when(jnp.logical_and(nx > 0, nx <= CAP))
        def _():
            pltpu.make_async_copy(
                tab8_hbm.at[idx2_v[slot].at[pl.ds(0, 12 * CAP)]],
                rows2_v[slot].at[pl.ds(0, 12 * CAP)], sem2[slot]).wait()

        @pl.when(nx > CAP)
        def _():
            pltpu.make_async_copy(
                tab8_hbm.at[idx2_v[slot]], rows2_v[slot], sem2[slot]).wait()

    def do_comb(i, slot, nx):
        def comb_body(g, _):
            g16 = g * 16
            p = g16 + lanes
            cl = cl_v[slot, pl.ds(g16, 16)]
            clh = (cl + 1) & 7
            tx = t_v[slot * 3, pl.ds(g16, 16)]
            ty = t_v[slot * 3 + 1, pl.ds(g16, 16)]
            tz = t_v[slot * 3 + 2, pl.ds(g16, 16)]
            one = jnp.float32(1.0)
            wy = (one - ty, ty)
            wz = (one - tz, tz)
            acc = [None, None, None]
            for dz in range(2):
                for dy in range(2):
                    wzy = wz[dz] * wy[dy]
                    for ch in range(3):
                        m12 = (dz * 3 + ch) * 2 + dy
                        row = m12 * C + p
                        v_lo = plsc.load_gather(rows_v[slot], [row, cl])
                        v_hi = plsc.load_gather(rows_v[slot], [row, clh])
                        xv = v_lo + tx * (v_hi - v_lo)
                        acc[ch] = (wzy * xv if acc[ch] is None
                                   else acc[ch] + wzy * xv)
            for ch in range(3):
                out_v[ch][pl.ds(g16, 16)] = acc[ch]
            return 0

        lax.fori_loop(0, G, comb_body, 0)

        def fix_body(g, _):
            r = jnp.minimum(g * 16 + lanes, C - 1)
            msk = (g * 16 + lanes) < nx
            pid = plsc.load_gather(xid_v[slot], [r])
            tx = plsc.load_gather(t_v, [jnp.full((16,), slot * 3,
                                                 jnp.int32), pid])
            ty = plsc.load_gather(t_v, [jnp.full((16,), slot * 3 + 1,
                                                 jnp.int32), pid])
            tz = plsc.load_gather(t_v, [jnp.full((16,), slot * 3 + 2,
                                                 jnp.int32), pid])
            one = jnp.float32(1.0)
            wy = (one - ty, ty)
            wz = (one - tz, tz)
            col7 = jnp.full((16,), 7, jnp.int32)
            col0 = jnp.full((16,), 0, jnp.int32)
            r12 = r * 12
            acc = [None, None, None]
            for dz in range(2):
                for dy in range(2):
                    wzy = wz[dz] * wy[dy]
                    for ch in range(3):
                        m12 = (dz * 3 + ch) * 2 + dy
                        v_lo = plsc.load_gather(
                            rows_v[slot], [m12 * C + pid, col7])
                        v_hi = plsc.load_gather(
                            rows2_v[slot], [r12 + m12, col0])
                        xv = v_lo + tx * (v_hi - v_lo)
                        acc[ch] = (wzy * xv if acc[ch] is None
                                   else acc[ch] + wzy * xv)
            for ch in range(3):
                plsc.store_scatter(out_v[ch], [pid], acc[ch], mask=msk)
            return 0

        lax.fori_loop(0, (nx + 15) >> 4, fix_body, 0)

        base = wid * P + i * C
        for ch in range(3):
            pltpu.sync_copy(out_v[ch], o_hbm[ch].at[pl.ds(base, C)])

    load_pts(0, 0)
    nx0 = do_idx(0, 0)
    fire(0, nx0)

    def pair_body(j, nxs):
        nx0, _ = nxs
        i0 = j * 2
        load_pts(i0 + 1, 1)
        nx1 = do_idx(i0 + 1, 1)
        fire(1, nx1)
        wait_gather(0, nx0)
        do_comb(i0, 0, nx0)

        def idx_and_fire():
            load_pts(i0 + 2, 0)
            nxn = do_idx(i0 + 2, 0)
            fire(0, nxn)
            return nxn

        nx0n = lax.cond(i0 + 2 < NCHUNK, idx_and_fire,
                        lambda: jnp.int32(0))

        wait_gather(1, nx1)
        do_comb(i0 + 1, 1, nx1)
        return (nx0n, jnp.int32(0))

    lax.fori_loop(0, NCHUNK // 2, pair_body,
                  (nx0, jnp.int32(0)))


def kernel(input, feature_params):
    tab8 = (feature_params.transpose(0, 3, 1, 2)
            .reshape(RES, 3, 32, 8, 2, 128)
            .transpose(0, 1, 2, 4, 3, 5)
            .reshape(NROW8, 8))
    xs = input[:, 0]
    ys = input[:, 1]
    zs = input[:, 2]
    o0, o1, o2 = _trilerp(xs, ys, zs, tab8)
    return jnp.stack([o0, o1, o2], axis=1)


# speedup vs baseline: 5.6443x; 5.6443x over previous
import functools

import jax
import jax.numpy as jnp
from jax import lax
from jax.experimental import pallas as pl
from jax.experimental.pallas import tpu as pltpu
from jax.experimental.pallas import tpu_sc as plsc

RES = 256
N = 1048576
NW = 32
P = N // NW
C = 256
G = C // 16
NCHUNK = P // C
NROW8 = RES * RES * RES * 3 // 8

_mesh = plsc.VectorSubcoreMesh(core_axis_name="c", subcore_axis_name="s")


@functools.partial(
    pl.kernel,
    mesh=_mesh,
    out_type=tuple(jax.ShapeDtypeStruct((N,), jnp.float32) for _ in range(3)),
    compiler_params=pltpu.CompilerParams(
        use_tc_tiling_on_sc=False, needs_layout_passes=False),
    scratch_types=[
        tuple(tuple(pltpu.VMEM((C,), jnp.float32) for _ in range(3))
              for _ in range(2)),
        tuple(pltpu.VMEM((24 * C,), jnp.int32) for _ in range(2)),
        pltpu.VMEM((4, C), jnp.int32),
        tuple(pltpu.VMEM((24 * C, 8), jnp.float32) for _ in range(2)),
        pltpu.VMEM((6, C), jnp.float32),
        tuple(pltpu.VMEM((C,), jnp.float32) for _ in range(3)),
        tuple(pltpu.SemaphoreType.DMA for _ in range(2)),
    ],
)
def _trilerp(xs_hbm, ys_hbm, zs_hbm, tab8_hbm, ox_hbm, oy_hbm, oz_hbm,
             pts_v, idx_v, cl_v, rows_v, t_v, out_v, sem):
    wid = lax.axis_index("s") * 2 + lax.axis_index("c")
    lanes = lax.iota(jnp.int32, 16)
    coord_hbm = (xs_hbm, ys_hbm, zs_hbm)
    o_hbm = (ox_hbm, oy_hbm, oz_hbm)

    def load_pts(i, slot):
        base = wid * P + i * C
        for ch in range(3):
            pltpu.sync_copy(coord_hbm[ch].at[pl.ds(base, C)],
                            pts_v[slot][ch])

    def do_idx(i, slot):
        def idx_body(g, _):
            g16 = g * 16
            lo = []
            for ch in range(3):
                coord = pts_v[slot][ch][pl.ds(g16, 16)]
                s = coord * jnp.float32(RES - 1)
                li = jnp.minimum(s.astype(jnp.int32), RES - 2)
                t_v[slot * 3 + ch, pl.ds(g16, 16)] = (
                    s - li.astype(jnp.float32))
                lo.append(li)
            ix, iy, iz = lo
            izc = iz * 3
            iy1 = iy + 1
            ix1 = ix + 1
            yt = (((iy >> 3) << 8) + ((iy & 7) << 4),
                  ((iy1 >> 3) << 8) + ((iy1 & 7) << 4))
            xt_lo = ((ix >> 7) << 7) + ((ix >> 3) & 15)
            xt_hi = ((ix1 >> 7) << 7) + ((ix1 >> 3) & 15)
            cl_v[slot * 2, pl.ds(g16, 16)] = ix & 7
            cl_v[slot * 2 + 1, pl.ds(g16, 16)] = ix1 & 7
            yx = ((yt[0] + xt_lo, yt[0] + xt_hi),
                  (yt[1] + xt_lo, yt[1] + xt_hi))
            for dz in range(2):
                for ch in range(3):
                    zterm = (izc + (dz * 3 + ch)) << 13
                    for dy in range(2):
                        m12 = (dz * 3 + ch) * 2 + dy
                        q = (m12 * C + g16) * 2 + lanes * 2
                        plsc.store_scatter(idx_v[slot], [q],
                                           zterm + yx[dy][0])
                        plsc.store_scatter(idx_v[slot], [q + 1],
                                           zterm + yx[dy][1])
            return 0

        lax.fori_loop(0, G, idx_body, 0)

    def fire(slot):
        pltpu.async_copy(tab8_hbm.at[idx_v[slot]], rows_v[slot], sem[slot])

    def wait_gather(slot):
        pltpu.make_async_copy(
            tab8_hbm.at[idx_v[slot]], rows_v[slot], sem[slot]).wait()

    def do_comb(i, slot):
        def comb_body(g, _):
            g16 = g * 16
            p = g16 + lanes
            cl = cl_v[slot * 2, pl.ds(g16, 16)]
            ch_ = cl_v[slot * 2 + 1, pl.ds(g16, 16)]
            tx = t_v[slot * 3, pl.ds(g16, 16)]
            ty = t_v[slot * 3 + 1, pl.ds(g16, 16)]
            tz = t_v[slot * 3 + 2, pl.ds(g16, 16)]
            one = jnp.float32(1.0)
            wy = (one - ty, ty)
            wz = (one - tz, tz)
            acc = [None, None, None]
            for dz in range(2):
                for dy in range(2):
                    wzy = wz[dz] * wy[dy]
                    for ch in range(3):
                        m12 = (dz * 3 + ch) * 2 + dy
                        q = (m12 * C + g16) * 2 + lanes * 2
                        v_lo = plsc.load_gather(rows_v[slot], [q, cl])
                        v_hi = plsc.load_gather(rows_v[slot], [q + 1, ch_])
                        xv = v_lo + tx * (v_hi - v_lo)
                        acc[ch] = (wzy * xv if acc[ch] is None
                                   else acc[ch] + wzy * xv)
            for ch in range(3):
                out_v[ch][pl.ds(g16, 16)] = acc[ch]
            return 0

        lax.fori_loop(0, G, comb_body, 0)
        base = wid * P + i * C
        for ch in range(3):
            pltpu.sync_copy(out_v[ch], o_hbm[ch].at[pl.ds(base, C)])

    load_pts(0, 0)
    do_idx(0, 0)
    fire(0)

    def pair_body(j, _):
        i0 = j * 2
        load_pts(i0 + 1, 1)
        do_idx(i0 + 1, 1)
        fire(1)
        wait_gather(0)
        do_comb(i0, 0)

        @pl.when(i0 + 2 < NCHUNK)
        def _():
            load_pts(i0 + 2, 0)
            do_idx(i0 + 2, 0)
            fire(0)

        wait_gather(1)
        do_comb(i0 + 1, 1)
        return 0

    lax.fori_loop(0, NCHUNK // 2, pair_body, 0)


def kernel(input, feature_params):
    tab8 = (feature_params.transpose(0, 3, 1, 2)
            .reshape(RES, 3, 32, 8, 2, 128)
            .transpose(0, 1, 2, 4, 3, 5)
            .reshape(NROW8, 8))
    xs = input[:, 0]
    ys = input[:, 1]
    zs = input[:, 2]
    o0, o1, o2 = _trilerp(xs, ys, zs, tab8)
    return jnp.stack([o0, o1, o2], axis=1)
